# baseline (speedup 1.0000x reference)
import functools

import jax
import jax.numpy as jnp
from jax import lax
from jax.experimental import pallas as pl
from jax.experimental.pallas import tpu as pltpu
from jax.experimental.pallas import tpu_sc as plsc

NC = 2
NS = 16
NW = NC * NS
CHUNK = 128
ZROWS = 32
NBUF = 2


def _round_up(x, m):
    return (x + m - 1) // m * m


@functools.lru_cache(maxsize=None)
def _make_segsum(nq_pad, nd_pad, eq_pad, ed_pad, d):
    cq = eq_pad // (NW * CHUNK)
    cd = ed_pad // (NW * CHUNK)
    rq = nq_pad // NS
    rd = nd_pad // NS
    mesh = plsc.VectorSubcoreMesh(core_axis_name="c", subcore_axis_name="s")

    @functools.partial(
        pl.kernel,
        out_type=[
            jax.ShapeDtypeStruct((NC, nq_pad, d), jnp.float32),
            jax.ShapeDtypeStruct((NC, nd_pad, d), jnp.float32),
        ],
        mesh=mesh,
        scratch_types=(
            [pltpu.VMEM((2, CHUNK), jnp.int32) for _ in range(NBUF)]
            + [pltpu.VMEM((CHUNK, d), jnp.float32) for _ in range(NBUF)]
            + [
                pltpu.VMEM((ZROWS, d), jnp.float32),
                pltpu.VMEM_SHARED((nq_pad, d), jnp.float32),
                pltpu.VMEM_SHARED((nd_pad, d), jnp.float32),
                pltpu.SemaphoreType.DMA((NBUF,)),
                pltpu.SemaphoreType.DMA((NBUF,)),
                pltpu.SemaphoreType.DMA((NBUF,)),
            ]
        ),
    )
    def seg(xq_hbm, xd_hbm, qe_hbm, de_hbm, outq_hbm, outd_hbm,
            i0, i1, r0, r1, zbuf, accq, accd, isem, gsem, ssem):
        idxb = [i0, i1]
        rowb = [r0, r1]
        c = lax.axis_index("c")
        s = lax.axis_index("s")
        wid = s * NC + c

        @pl.loop(0, ZROWS)
        def _(r):
            @pl.loop(0, d // 16)
            def _(k):
                zbuf[r, pl.ds(k * 16, 16)] = jnp.zeros((16,), jnp.float32)

        def zero_rows(acc, base, total):
            chunk = min(total, ZROWS)
            nloop = -(-total // chunk)
            last = total - chunk

            @pl.loop(0, nloop)
            def _(j):
                off = jnp.minimum(j * chunk, last)
                pltpu.sync_copy(zbuf.at[pl.ds(0, chunk)],
                                acc.at[pl.ds(base + off, chunk)])

        zero_rows(accq, s * rq, rq)
        zero_rows(accd, s * rd, rd)

        plsc.subcore_barrier()

        def ring(e_hbm, x_hbm, acc, nchunks):
            def idx_cp(b, chunk):
                return pltpu.make_async_copy(e_hbm.at[chunk], idxb[b],
                                             isem.at[b])

            def gat_cp(b):
                return pltpu.make_async_copy(x_hbm.at[idxb[b].at[0]], rowb[b],
                                             gsem.at[b])

            def sct_cp(b):
                return pltpu.make_async_copy(rowb[b], acc.at[idxb[b].at[1]],
                                             ssem.at[b])

            base = wid * nchunks

            @pl.loop(0, nchunks // NBUF)
            def _(g):
                for b in range(NBUF):
                    @pl.when(g > 0)
                    def _():
                        sct_cp(b).wait()
                    idx_cp(b, base + g * NBUF + b).start()
                for b in range(NBUF):
                    idx_cp(b, base + g * NBUF + b).wait()
                    gat_cp(b).start()
                for b in range(NBUF):
                    gat_cp(b).wait()
                    sct_cp(b).start(add=True)

            for b in range(NBUF):
                sct_cp(b).wait()

        ring(qe_hbm, xq_hbm, accq, cq)
        ring(de_hbm, xd_hbm, accd, cd)

        plsc.subcore_barrier()

        pltpu.sync_copy(accq.at[pl.ds(s * rq, rq)],
                        outq_hbm.at[c].at[pl.ds(s * rq, rq)])
        pltpu.sync_copy(accd.at[pl.ds(s * rd, rd)],
                        outd_hbm.at[c].at[pl.ds(s * rd, rd)])

    return seg


def _mm_body(x_ref, p0_ref, p1_ref, w_ref, b_ref, o_ref):
    acc = x_ref[...] + p0_ref[...] + p1_ref[...]
    y = jnp.dot(acc, w_ref[...], preferred_element_type=jnp.float32,
                 precision=lax.Precision.HIGHEST) + b_ref[...]
    o_ref[...] = jnp.maximum(y, 0.0)


def _mm_add_relu(x, p, w, b, bm):
    m, k = x.shape
    n = w.shape[1]
    return pl.pallas_call(
        _mm_body,
        grid=(m // bm,),
        in_specs=[
            pl.BlockSpec((bm, k), lambda i: (i, 0)),
            pl.BlockSpec((bm, k), lambda i: (i, 0)),
            pl.BlockSpec((bm, k), lambda i: (i, 0)),
            pl.BlockSpec((k, n), lambda i: (0, 0)),
            pl.BlockSpec((1, n), lambda i: (0, 0)),
        ],
        out_specs=pl.BlockSpec((bm, n), lambda i: (i, 0)),
        out_shape=jax.ShapeDtypeStruct((m, n), jnp.float32),
    )(x, p[0], p[1], w, b.reshape(1, n))


def _mm_plain_body(x_ref, w_ref, o_ref):
    o_ref[...] = jnp.dot(x_ref[...], w_ref[...],
                         preferred_element_type=jnp.float32,
                 precision=lax.Precision.HIGHEST)


def _mm_plain(x, w, bm):
    m, k = x.shape
    n = w.shape[1]
    return pl.pallas_call(
        _mm_plain_body,
        grid=(m // bm,),
        in_specs=[
            pl.BlockSpec((bm, k), lambda i: (i, 0)),
            pl.BlockSpec((k, n), lambda i: (0, 0)),
        ],
        out_specs=pl.BlockSpec((bm, n), lambda i: (i, 0)),
        out_shape=jax.ShapeDtypeStruct((m, n), jnp.float32),
    )(x, w)


def _reduce_body(u_ref, p0_ref, p1_ref, b_ref, o_ref):
    i = pl.program_id(0)
    h = jnp.maximum(u_ref[...] + p0_ref[...] + p1_ref[...] + b_ref[...], 0.0)

    @pl.when(i == 0)
    def _():
        o_ref[...] = jnp.zeros_like(o_ref)

    o_ref[...] += jnp.sum(h, axis=0, keepdims=True)


def _reduce_relu_sum(u, p, b, bm):
    m, n = u.shape
    return pl.pallas_call(
        _reduce_body,
        grid=(m // bm,),
        in_specs=[
            pl.BlockSpec((bm, n), lambda i: (i, 0)),
            pl.BlockSpec((bm, n), lambda i: (i, 0)),
            pl.BlockSpec((bm, n), lambda i: (i, 0)),
            pl.BlockSpec((1, n), lambda i: (0, 0)),
        ],
        out_specs=pl.BlockSpec((1, n), lambda i: (0, 0)),
        out_shape=jax.ShapeDtypeStruct((1, n), jnp.float32),
    )(u, p[0], p[1], b.reshape(1, n))


def _head_body(gq_ref, gd_ref, w1q_ref, w1d_ref, b1_ref, w2_ref, b2_ref,
               w3_ref, b3_ref, o_ref):
    z1 = jnp.maximum(
        jnp.dot(gq_ref[...], w1q_ref[...], preferred_element_type=jnp.float32,
                 precision=lax.Precision.HIGHEST)
        + jnp.dot(gd_ref[...], w1d_ref[...], preferred_element_type=jnp.float32,
                 precision=lax.Precision.HIGHEST)
        + b1_ref[...], 0.0)
    z2 = jnp.maximum(
        jnp.dot(z1, w2_ref[...], preferred_element_type=jnp.float32,
                 precision=lax.Precision.HIGHEST)
        + b2_ref[...], 0.0)
    z3 = jnp.maximum(jnp.sum(z2 * w3_ref[...], axis=1, keepdims=True)
                     + b3_ref[...], 0.0)
    o_ref[...] = z3


def _head(gq, gd, w1, b1, w2, b2, w3, b3):
    o1 = w1.shape[1]
    h2 = w2.shape[1]
    k = gq.shape[1]
    return pl.pallas_call(
        _head_body,
        out_shape=jax.ShapeDtypeStruct((1, 1), jnp.float32),
    )(gq, gd, w1[:k], w1[k:], b1.reshape(1, o1), w2, b2.reshape(1, h2),
      w3.reshape(1, h2), b3.reshape(1, 1))


def _pad_edges(edge_index, e_pad, dump):
    src = edge_index[0].astype(jnp.int32)
    dst = edge_index[1].astype(jnp.int32)
    e = src.shape[0]
    pad = e_pad - e
    src_p = jnp.concatenate([src, jnp.zeros((pad,), jnp.int32)])
    dst_p = jnp.concatenate([dst, jnp.full((pad,), dump, jnp.int32)])
    return jnp.stack([src_p.reshape(-1, CHUNK), dst_p.reshape(-1, CHUNK)],
                     axis=1)


def kernel(query_x, query_edge_index, data_x, data_edge_index,
           Wq1, bq1, Wq2, bq2, Wd1, bd1, Wd2, bd2,
           L1W, L1b, L2W, L2b, L3W, L3b):
    nq, d = query_x.shape
    nd = data_x.shape[0]
    eq = query_edge_index.shape[1]
    ed = data_edge_index.shape[1]

    nq_pad = _round_up(nq + 1, NS * 8)
    nd_pad = _round_up(nd + 1, NS * 8)
    eq_pad = _round_up(eq, NW * CHUNK * NBUF)
    ed_pad = _round_up(ed, NW * CHUNK * NBUF)

    qe = _pad_edges(query_edge_index, eq_pad, nq)
    de = _pad_edges(data_edge_index, ed_pad, nd)

    seg = _make_segsum(nq_pad, nd_pad, eq_pad, ed_pad, d)

    pq1, pd1 = seg(query_x, data_x, qe, de)
    hq = _mm_add_relu(query_x, pq1, Wq1, bq1, bm=nq)
    hd = _mm_add_relu(data_x, pd1, Wd1, bd1, bm=1000)

    uq = _mm_plain(hq, Wq2, bm=nq)
    ud = _mm_plain(hd, Wd2, bm=1000)
    pq2, pd2 = seg(uq, ud, qe, de)

    gq = _reduce_relu_sum(uq, pq2, bq2, bm=nq)
    gd = _reduce_relu_sum(ud, pd2, bd2, bm=1000)

    o = _head(gq, gd, L1W, L1b, L2W, L2b, L3W, L3b)
    return o.reshape(1)


# speedup vs baseline: 1.0202x; 1.0202x over previous
import functools

import jax
import jax.numpy as jnp
from jax import lax
from jax.experimental import pallas as pl
from jax.experimental.pallas import tpu as pltpu
from jax.experimental.pallas import tpu_sc as plsc

NC = 2
NS = 16
NW = NC * NS
CHUNK = 128
ZROWS = 32
NIDX = 4
NROW = 2
NBUF = NIDX


def _round_up(x, m):
    return (x + m - 1) // m * m


@functools.lru_cache(maxsize=None)
def _make_segsum(nq_pad, nd_pad, eq_pad, ed_pad, d):
    cq = eq_pad // (NW * CHUNK)
    cd = ed_pad // (NW * CHUNK)
    rq = nq_pad // NS
    rd = nd_pad // NS
    mesh = plsc.VectorSubcoreMesh(core_axis_name="c", subcore_axis_name="s")

    @functools.partial(
        pl.kernel,
        out_type=[
            jax.ShapeDtypeStruct((NC, nq_pad, d), jnp.float32),
            jax.ShapeDtypeStruct((NC, nd_pad, d), jnp.float32),
        ],
        mesh=mesh,
        scratch_types=(
            [pltpu.VMEM((2, CHUNK), jnp.int32) for _ in range(NIDX)]
            + [pltpu.VMEM((CHUNK, d), jnp.float32) for _ in range(NROW)]
            + [
                pltpu.VMEM((ZROWS, d), jnp.float32),
                pltpu.VMEM_SHARED((nq_pad, d), jnp.float32),
                pltpu.VMEM_SHARED((nd_pad, d), jnp.float32),
                pltpu.SemaphoreType.DMA((NIDX,)),
                pltpu.SemaphoreType.DMA((NROW,)),
                pltpu.SemaphoreType.DMA((NROW,)),
            ]
        ),
    )
    def seg(xq_hbm, xd_hbm, qe_hbm, de_hbm, outq_hbm, outd_hbm,
            i0, i1, i2, i3, r0, r1, zbuf, accq, accd, isem, gsem, ssem):
        idxb = [i0, i1, i2, i3]
        rowb = [r0, r1]
        c = lax.axis_index("c")
        s = lax.axis_index("s")
        wid = s * NC + c

        @pl.loop(0, ZROWS)
        def _(r):
            @pl.loop(0, d // 16)
            def _(k):
                zbuf[r, pl.ds(k * 16, 16)] = jnp.zeros((16,), jnp.float32)

        def zero_rows(acc, base, total):
            chunk = min(total, ZROWS)
            nloop = -(-total // chunk)
            last = total - chunk

            @pl.loop(0, nloop)
            def _(j):
                off = jnp.minimum(j * chunk, last)
                pltpu.sync_copy(zbuf.at[pl.ds(0, chunk)],
                                acc.at[pl.ds(base + off, chunk)])

        zero_rows(accq, s * rq, rq)
        zero_rows(accd, s * rd, rd)

        plsc.subcore_barrier()

        def ring(e_hbm, x_hbm, acc, nchunks):
            ngroups = nchunks // NIDX

            def icp(bi, chunk):
                return pltpu.make_async_copy(e_hbm.at[chunk], idxb[bi],
                                             isem.at[bi])

            def gcp(br, bi):
                return pltpu.make_async_copy(x_hbm.at[idxb[bi].at[0]],
                                             rowb[br], gsem.at[br])

            def scp(br, bi):
                return pltpu.make_async_copy(rowb[br], acc.at[idxb[bi].at[1]],
                                             ssem.at[br])

            base = wid * nchunks
            icp(0, base).start()
            icp(1, base + 1).start()

            @pl.loop(0, ngroups)
            def _(g):
                for b in range(NIDX):
                    br = b % NROW
                    bp = (b + 2) % NIDX
                    if b >= 2:
                        scp(br, bp).wait()
                    else:
                        @pl.when(g > 0)
                        def _():
                            scp(br, bp).wait()
                    if b < 2:
                        icp(bp, base + g * NIDX + b + 2).start()
                    else:
                        @pl.when(g < ngroups - 1)
                        def _():
                            icp(bp, base + g * NIDX + b + 2).start()
                    icp(b, base + g * NIDX + b).wait()
                    gcp(br, b).start()
                    gcp(br, b).wait()
                    scp(br, b).start(add=True)

            scp(0, 2).wait()
            scp(1, 3).wait()

        ring(qe_hbm, xq_hbm, accq, cq)
        ring(de_hbm, xd_hbm, accd, cd)

        plsc.subcore_barrier()

        pltpu.sync_copy(accq.at[pl.ds(s * rq, rq)],
                        outq_hbm.at[c].at[pl.ds(s * rq, rq)])
        pltpu.sync_copy(accd.at[pl.ds(s * rd, rd)],
                        outd_hbm.at[c].at[pl.ds(s * rd, rd)])

    return seg


def _mm_body(x_ref, p0_ref, p1_ref, w_ref, b_ref, o_ref):
    acc = x_ref[...] + p0_ref[...] + p1_ref[...]
    y = jnp.dot(acc, w_ref[...], preferred_element_type=jnp.float32,
                 precision=lax.Precision.HIGHEST) + b_ref[...]
    o_ref[...] = jnp.maximum(y, 0.0)


def _mm_add_relu(x, p, w, b, bm):
    m, k = x.shape
    n = w.shape[1]
    return pl.pallas_call(
        _mm_body,
        grid=(m // bm,),
        in_specs=[
            pl.BlockSpec((bm, k), lambda i: (i, 0)),
            pl.BlockSpec((bm, k), lambda i: (i, 0)),
            pl.BlockSpec((bm, k), lambda i: (i, 0)),
            pl.BlockSpec((k, n), lambda i: (0, 0)),
            pl.BlockSpec((1, n), lambda i: (0, 0)),
        ],
        out_specs=pl.BlockSpec((bm, n), lambda i: (i, 0)),
        out_shape=jax.ShapeDtypeStruct((m, n), jnp.float32),
    )(x, p[0], p[1], w, b.reshape(1, n))


def _mm_plain_body(x_ref, w_ref, o_ref):
    o_ref[...] = jnp.dot(x_ref[...], w_ref[...],
                         preferred_element_type=jnp.float32,
                 precision=lax.Precision.HIGHEST)


def _mm_plain(x, w, bm):
    m, k = x.shape
    n = w.shape[1]
    return pl.pallas_call(
        _mm_plain_body,
        grid=(m // bm,),
        in_specs=[
            pl.BlockSpec((bm, k), lambda i: (i, 0)),
            pl.BlockSpec((k, n), lambda i: (0, 0)),
        ],
        out_specs=pl.BlockSpec((bm, n), lambda i: (i, 0)),
        out_shape=jax.ShapeDtypeStruct((m, n), jnp.float32),
    )(x, w)


def _reduce_body(u_ref, p0_ref, p1_ref, b_ref, o_ref):
    i = pl.program_id(0)
    h = jnp.maximum(u_ref[...] + p0_ref[...] + p1_ref[...] + b_ref[...], 0.0)

    @pl.when(i == 0)
    def _():
        o_ref[...] = jnp.zeros_like(o_ref)

    o_ref[...] += jnp.sum(h, axis=0, keepdims=True)


def _reduce_relu_sum(u, p, b, bm):
    m, n = u.shape
    return pl.pallas_call(
        _reduce_body,
        grid=(m // bm,),
        in_specs=[
            pl.BlockSpec((bm, n), lambda i: (i, 0)),
            pl.BlockSpec((bm, n), lambda i: (i, 0)),
            pl.BlockSpec((bm, n), lambda i: (i, 0)),
            pl.BlockSpec((1, n), lambda i: (0, 0)),
        ],
        out_specs=pl.BlockSpec((1, n), lambda i: (0, 0)),
        out_shape=jax.ShapeDtypeStruct((1, n), jnp.float32),
    )(u, p[0], p[1], b.reshape(1, n))


def _head_body(gq_ref, gd_ref, w1q_ref, w1d_ref, b1_ref, w2_ref, b2_ref,
               w3_ref, b3_ref, o_ref):
    z1 = jnp.maximum(
        jnp.dot(gq_ref[...], w1q_ref[...], preferred_element_type=jnp.float32,
                 precision=lax.Precision.HIGHEST)
        + jnp.dot(gd_ref[...], w1d_ref[...], preferred_element_type=jnp.float32,
                 precision=lax.Precision.HIGHEST)
        + b1_ref[...], 0.0)
    z2 = jnp.maximum(
        jnp.dot(z1, w2_ref[...], preferred_element_type=jnp.float32,
                 precision=lax.Precision.HIGHEST)
        + b2_ref[...], 0.0)
    z3 = jnp.maximum(jnp.sum(z2 * w3_ref[...], axis=1, keepdims=True)
                     + b3_ref[...], 0.0)
    o_ref[...] = z3


def _head(gq, gd, w1, b1, w2, b2, w3, b3):
    o1 = w1.shape[1]
    h2 = w2.shape[1]
    k = gq.shape[1]
    return pl.pallas_call(
        _head_body,
        out_shape=jax.ShapeDtypeStruct((1, 1), jnp.float32),
    )(gq, gd, w1[:k], w1[k:], b1.reshape(1, o1), w2, b2.reshape(1, h2),
      w3.reshape(1, h2), b3.reshape(1, 1))


def _pad_edges(edge_index, e_pad, dump):
    src = edge_index[0].astype(jnp.int32)
    dst = edge_index[1].astype(jnp.int32)
    e = src.shape[0]
    pad = e_pad - e
    src_p = jnp.concatenate([src, jnp.zeros((pad,), jnp.int32)])
    dst_p = jnp.concatenate([dst, jnp.full((pad,), dump, jnp.int32)])
    return jnp.stack([src_p.reshape(-1, CHUNK), dst_p.reshape(-1, CHUNK)],
                     axis=1)


def kernel(query_x, query_edge_index, data_x, data_edge_index,
           Wq1, bq1, Wq2, bq2, Wd1, bd1, Wd2, bd2,
           L1W, L1b, L2W, L2b, L3W, L3b):
    nq, d = query_x.shape
    nd = data_x.shape[0]
    eq = query_edge_index.shape[1]
    ed = data_edge_index.shape[1]

    nq_pad = _round_up(nq + 1, NS * 8)
    nd_pad = _round_up(nd + 1, NS * 8)
    eq_pad = _round_up(eq, NW * CHUNK * NBUF)
    ed_pad = _round_up(ed, NW * CHUNK * NBUF)

    qe = _pad_edges(query_edge_index, eq_pad, nq)
    de = _pad_edges(data_edge_index, ed_pad, nd)

    seg = _make_segsum(nq_pad, nd_pad, eq_pad, ed_pad, d)

    pq1, pd1 = seg(query_x, data_x, qe, de)
    hq = _mm_add_relu(query_x, pq1, Wq1, bq1, bm=nq)
    hd = _mm_add_relu(data_x, pd1, Wd1, bd1, bm=1000)

    uq = _mm_plain(hq, Wq2, bm=nq)
    ud = _mm_plain(hd, Wd2, bm=1000)
    pq2, pd2 = seg(uq, ud, qe, de)

    gq = _reduce_relu_sum(uq, pq2, bq2, bm=nq)
    gd = _reduce_relu_sum(ud, pd2, bd2, bm=1000)

    o = _head(gq, gd, L1W, L1b, L2W, L2b, L3W, L3b)
    return o.reshape(1)


# speedup vs baseline: 1.3103x; 1.2843x over previous
import functools

import jax
import jax.numpy as jnp
from jax import lax
from jax.experimental import pallas as pl
from jax.experimental.pallas import tpu as pltpu
from jax.experimental.pallas import tpu_sc as plsc

NC = 2
NS = 16
NW = NC * NS
CHUNK = 128
ZROWS = 32
NIDX = 4
NROW = 2
NBUF = NIDX


def _round_up(x, m):
    return (x + m - 1) // m * m


@functools.lru_cache(maxsize=None)
def _make_segsum(nq_pad, nd_pad, eq_pad, ed_pad, d):
    cq = eq_pad // (NW * CHUNK)
    cd = ed_pad // (NW * CHUNK)
    rq = nq_pad // NS
    rd = nd_pad // NS
    mesh = plsc.VectorSubcoreMesh(core_axis_name="c", subcore_axis_name="s")

    @functools.partial(
        pl.kernel,
        out_type=[
            jax.ShapeDtypeStruct((NC, nq_pad, d), jnp.float32),
            jax.ShapeDtypeStruct((NC, nd_pad, d), jnp.float32),
        ],
        mesh=mesh,
        scratch_types=(
            [pltpu.VMEM((2, CHUNK), jnp.int32) for _ in range(NIDX)]
            + [pltpu.VMEM((CHUNK, d), jnp.float32) for _ in range(NROW)]
            + [
                pltpu.VMEM((ZROWS, d), jnp.float32),
                pltpu.VMEM_SHARED((nq_pad, d), jnp.float32),
                pltpu.VMEM_SHARED((nd_pad, d), jnp.float32),
                pltpu.SemaphoreType.DMA((NIDX,)),
                pltpu.SemaphoreType.DMA((NROW,)),
                pltpu.SemaphoreType.DMA((NROW,)),
            ]
        ),
    )
    def seg(xq_hbm, xd_hbm, qe_hbm, de_hbm, outq_hbm, outd_hbm,
            i0, i1, i2, i3, r0, r1, zbuf, accq, accd, isem, gsem, ssem):
        idxb = [i0, i1, i2, i3]
        rowb = [r0, r1]
        c = lax.axis_index("c")
        s = lax.axis_index("s")
        wid = s * NC + c

        @pl.loop(0, ZROWS)
        def _(r):
            @pl.loop(0, d // 16)
            def _(k):
                zbuf[r, pl.ds(k * 16, 16)] = jnp.zeros((16,), jnp.float32)

        def zero_rows(acc, base, total):
            chunk = min(total, ZROWS)
            nloop = -(-total // chunk)
            last = total - chunk

            @pl.loop(0, nloop)
            def _(j):
                off = jnp.minimum(j * chunk, last)
                pltpu.sync_copy(zbuf.at[pl.ds(0, chunk)],
                                acc.at[pl.ds(base + off, chunk)])

        zero_rows(accq, s * rq, rq)
        zero_rows(accd, s * rd, rd)

        plsc.subcore_barrier()

        def ring(e_hbm, x_hbm, acc, nchunks):
            ngroups = nchunks // NIDX

            def icp(bi, chunk):
                return pltpu.make_async_copy(e_hbm.at[chunk], idxb[bi],
                                             isem.at[bi])

            def gcp(br, bi):
                return pltpu.make_async_copy(x_hbm.at[idxb[bi].at[0]],
                                             rowb[br], gsem.at[br])

            def scp(br, bi):
                return pltpu.make_async_copy(rowb[br], acc.at[idxb[bi].at[1]],
                                             ssem.at[br])

            def cid(i):
                return i * NW + wid

            icp(0, cid(0)).start()
            icp(1, cid(1)).start()

            @pl.loop(0, ngroups)
            def _(g):
                for b in range(NIDX):
                    br = b % NROW
                    bp = (b + 2) % NIDX
                    if b >= 2:
                        scp(br, bp).wait()
                    else:
                        @pl.when(g > 0)
                        def _():
                            scp(br, bp).wait()
                    if b < 2:
                        icp(bp, cid(g * NIDX + b + 2)).start()
                    else:
                        @pl.when(g < ngroups - 1)
                        def _():
                            icp(bp, cid(g * NIDX + b + 2)).start()
                    icp(b, cid(g * NIDX + b)).wait()
                    gcp(br, b).start()
                    gcp(br, b).wait()
                    scp(br, b).start(add=True)

            scp(0, 2).wait()
            scp(1, 3).wait()

        ring(qe_hbm, xq_hbm, accq, cq)
        ring(de_hbm, xd_hbm, accd, cd)

        plsc.subcore_barrier()

        pltpu.sync_copy(accq.at[pl.ds(s * rq, rq)],
                        outq_hbm.at[c].at[pl.ds(s * rq, rq)])
        pltpu.sync_copy(accd.at[pl.ds(s * rd, rd)],
                        outd_hbm.at[c].at[pl.ds(s * rd, rd)])

    return seg


def _mm_body(x_ref, p0_ref, p1_ref, w_ref, b_ref, o_ref):
    acc = x_ref[...] + p0_ref[...] + p1_ref[...]
    y = jnp.dot(acc, w_ref[...], preferred_element_type=jnp.float32,
                 precision=lax.Precision.HIGHEST) + b_ref[...]
    o_ref[...] = jnp.maximum(y, 0.0)


def _mm_add_relu(x, p, w, b, bm):
    m, k = x.shape
    n = w.shape[1]
    return pl.pallas_call(
        _mm_body,
        grid=(m // bm,),
        in_specs=[
            pl.BlockSpec((bm, k), lambda i: (i, 0)),
            pl.BlockSpec((bm, k), lambda i: (i, 0)),
            pl.BlockSpec((bm, k), lambda i: (i, 0)),
            pl.BlockSpec((k, n), lambda i: (0, 0)),
            pl.BlockSpec((1, n), lambda i: (0, 0)),
        ],
        out_specs=pl.BlockSpec((bm, n), lambda i: (i, 0)),
        out_shape=jax.ShapeDtypeStruct((m, n), jnp.float32),
    )(x, p[0], p[1], w, b.reshape(1, n))


def _mm_plain_body(x_ref, w_ref, o_ref):
    o_ref[...] = jnp.dot(x_ref[...], w_ref[...],
                         preferred_element_type=jnp.float32,
                 precision=lax.Precision.HIGHEST)


def _mm_plain(x, w, bm):
    m, k = x.shape
    n = w.shape[1]
    return pl.pallas_call(
        _mm_plain_body,
        grid=(m // bm,),
        in_specs=[
            pl.BlockSpec((bm, k), lambda i: (i, 0)),
            pl.BlockSpec((k, n), lambda i: (0, 0)),
        ],
        out_specs=pl.BlockSpec((bm, n), lambda i: (i, 0)),
        out_shape=jax.ShapeDtypeStruct((m, n), jnp.float32),
    )(x, w)


def _reduce_body(u_ref, p0_ref, p1_ref, b_ref, o_ref):
    i = pl.program_id(0)
    h = jnp.maximum(u_ref[...] + p0_ref[...] + p1_ref[...] + b_ref[...], 0.0)

    @pl.when(i == 0)
    def _():
        o_ref[...] = jnp.zeros_like(o_ref)

    o_ref[...] += jnp.sum(h, axis=0, keepdims=True)


def _reduce_relu_sum(u, p, b, bm):
    m, n = u.shape
    return pl.pallas_call(
        _reduce_body,
        grid=(m // bm,),
        in_specs=[
            pl.BlockSpec((bm, n), lambda i: (i, 0)),
            pl.BlockSpec((bm, n), lambda i: (i, 0)),
            pl.BlockSpec((bm, n), lambda i: (i, 0)),
            pl.BlockSpec((1, n), lambda i: (0, 0)),
        ],
        out_specs=pl.BlockSpec((1, n), lambda i: (0, 0)),
        out_shape=jax.ShapeDtypeStruct((1, n), jnp.float32),
    )(u, p[0], p[1], b.reshape(1, n))


def _head_body(gq_ref, gd_ref, w1q_ref, w1d_ref, b1_ref, w2_ref, b2_ref,
               w3_ref, b3_ref, o_ref):
    z1 = jnp.maximum(
        jnp.dot(gq_ref[...], w1q_ref[...], preferred_element_type=jnp.float32,
                 precision=lax.Precision.HIGHEST)
        + jnp.dot(gd_ref[...], w1d_ref[...], preferred_element_type=jnp.float32,
                 precision=lax.Precision.HIGHEST)
        + b1_ref[...], 0.0)
    z2 = jnp.maximum(
        jnp.dot(z1, w2_ref[...], preferred_element_type=jnp.float32,
                 precision=lax.Precision.HIGHEST)
        + b2_ref[...], 0.0)
    z3 = jnp.maximum(jnp.sum(z2 * w3_ref[...], axis=1, keepdims=True)
                     + b3_ref[...], 0.0)
    o_ref[...] = z3


def _head(gq, gd, w1, b1, w2, b2, w3, b3):
    o1 = w1.shape[1]
    h2 = w2.shape[1]
    k = gq.shape[1]
    return pl.pallas_call(
        _head_body,
        out_shape=jax.ShapeDtypeStruct((1, 1), jnp.float32),
    )(gq, gd, w1[:k], w1[k:], b1.reshape(1, o1), w2, b2.reshape(1, h2),
      w3.reshape(1, h2), b3.reshape(1, 1))


def _pad_edges(edge_index, e_pad, dump, ndump):
    src = edge_index[0].astype(jnp.int32)
    dst = edge_index[1].astype(jnp.int32)
    e = src.shape[0]
    pad = e_pad - e
    src_p = jnp.concatenate([src, jnp.zeros((pad,), jnp.int32)])
    dump_rows = dump + jnp.arange(pad, dtype=jnp.int32) % ndump
    dst_p = jnp.concatenate([dst, dump_rows])
    return jnp.stack([src_p.reshape(-1, CHUNK), dst_p.reshape(-1, CHUNK)],
                     axis=1)


def kernel(query_x, query_edge_index, data_x, data_edge_index,
           Wq1, bq1, Wq2, bq2, Wd1, bd1, Wd2, bd2,
           L1W, L1b, L2W, L2b, L3W, L3b):
    nq, d = query_x.shape
    nd = data_x.shape[0]
    eq = query_edge_index.shape[1]
    ed = data_edge_index.shape[1]

    nq_pad = _round_up(nq + 1, NS * 8)
    nd_pad = _round_up(nd + 1, NS * 8)
    eq_pad = _round_up(eq, NW * CHUNK * NBUF)
    ed_pad = _round_up(ed, NW * CHUNK * NBUF)

    qe = _pad_edges(query_edge_index, eq_pad, nq, nq_pad - nq)
    de = _pad_edges(data_edge_index, ed_pad, nd, nd_pad - nd)

    seg = _make_segsum(nq_pad, nd_pad, eq_pad, ed_pad, d)

    pq1, pd1 = seg(query_x, data_x, qe, de)
    hq = _mm_add_relu(query_x, pq1, Wq1, bq1, bm=nq)
    hd = _mm_add_relu(data_x, pd1, Wd1, bd1, bm=1000)

    uq = _mm_plain(hq, Wq2, bm=nq)
    ud = _mm_plain(hd, Wd2, bm=1000)
    pq2, pd2 = seg(uq, ud, qe, de)

    gq = _reduce_relu_sum(uq, pq2, bq2, bm=nq)
    gd = _reduce_relu_sum(ud, pd2, bd2, bm=1000)

    o = _head(gq, gd, L1W, L1b, L2W, L2b, L3W, L3b)
    return o.reshape(1)
